# baseline (speedup 1.0000x reference)
import jax
import jax.numpy as jnp
from jax import lax
from jax.experimental import pallas as pl
from jax.experimental.pallas import tpu as pltpu
from jax.experimental.pallas import tpu_sc as plsc

B, N0, C_IN = 2, 10000, 128
N1, N2 = 5000, 2500
E0, E1 = 160000, 80000
CH = 16
SUB = 128
K = 256
N0P = 10240
N1P, N2P = 6144, 4096
E0P, E1P = 163840, 81920

F32 = jnp.float32
I32 = jnp.int32


def _pre_body(x_ref, w1_ref, amp_ref, gw_ref, z_ref, u_ref, ssrc_ref, sdst_ref):
    x = x_ref[0]
    w1 = w1_ref[...]
    amp = amp_ref[...]
    gw = gw_ref[...]
    z_ref[0] = jnp.dot(x, w1[:C_IN], preferred_element_type=F32)
    u_ref[0] = jnp.dot(x * amp, w1[C_IN:], preferred_element_type=F32)
    ssrc_ref[0, 0] = jnp.sum(x * gw[0, C_IN:2 * C_IN][None, :], axis=1)
    sdst_ref[0, 0] = jnp.sum(x * gw[0, 0:C_IN][None, :], axis=1)


@jax.jit
def _pre(X, W1, amp1, gw1r):
    return pl.pallas_call(
        _pre_body,
        grid=(B,),
        in_specs=[
            pl.BlockSpec((1, N0, C_IN), lambda b: (b, 0, 0)),
            pl.BlockSpec((2 * C_IN, CH), lambda b: (0, 0)),
            pl.BlockSpec((1, C_IN), lambda b: (0, 0)),
            pl.BlockSpec((1, 2 * C_IN + 1), lambda b: (0, 0)),
        ],
        out_specs=[
            pl.BlockSpec((1, N0, CH), lambda b: (b, 0, 0)),
            pl.BlockSpec((1, N0, CH), lambda b: (b, 0, 0)),
            pl.BlockSpec((1, 1, N0), lambda b: (b, 0, 0)),
            pl.BlockSpec((1, 1, N0), lambda b: (b, 0, 0)),
        ],
        out_shape=[
            jax.ShapeDtypeStruct((B, N0, CH), F32),
            jax.ShapeDtypeStruct((B, N0, CH), F32),
            jax.ShapeDtypeStruct((B, 1, N0), F32),
            jax.ShapeDtypeStruct((B, 1, N0), F32),
        ],
    )(X, W1, amp1, gw1r)


def _make_sc_layer(mode, n_src, n_dst_pad, e_total):
    NPT = n_dst_pad // 16
    EPT = e_total // 16
    NCHUNK = EPT // K
    NSUB = K // SUB
    NGATE = K // 16
    mesh = plsc.VectorSubcoreMesh(core_axis_name="c", subcore_axis_name="s")

    if mode == 1:
        out_type = [
            jax.ShapeDtypeStruct((B, n_dst_pad, CH), F32),
            jax.ShapeDtypeStruct((B * n_dst_pad,), F32),
            jax.ShapeDtypeStruct((B * n_dst_pad,), F32),
        ]
    else:
        out_type = [
            jax.ShapeDtypeStruct((B, n_dst_pad, CH), F32),
            jax.ShapeDtypeStruct((B, n_dst_pad, CH), F32),
        ]

    scratch = [
        pltpu.VMEM_SHARED((n_dst_pad, 32), F32),
        pltpu.VMEM((n_src,), F32),
        pltpu.VMEM((n_src,), F32),
        pltpu.VMEM((n_dst_pad,), F32),
        pltpu.VMEM((n_dst_pad,), I32),
        pltpu.VMEM((EPT,), I32),
        pltpu.VMEM((EPT,), I32),
        pltpu.VMEM((EPT,), F32),
        pltpu.VMEM((K, CH), F32),
        pltpu.VMEM((K, CH), F32),
        pltpu.VMEM((K, 32), F32),
        pltpu.VMEM((K, 32), F32),
        pltpu.VMEM((NPT, 32), F32),
        pltpu.VMEM((NPT, CH), F32),
        pltpu.VMEM((NPT, CH), F32),
        pltpu.VMEM((NPT,), F32),
        pltpu.VMEM((NPT,), F32),
        pltpu.VMEM((16,), F32),
        pltpu.VMEM((16,), F32),
        pltpu.VMEM((16,), F32),
        pltpu.VMEM((16,), F32),
        pltpu.SemaphoreType.DMA,
        pltpu.SemaphoreType.DMA,
        pltpu.SemaphoreType.DMA,
        pltpu.SemaphoreType.DMA,
        pltpu.SemaphoreType.DMA,
    ]

    def body(V_h, Z_h, ssrcT, sdstT, resf_h, resi_h, src1_h, dst1_h, ef_h,
             consts_h, aux1_h, aux2_h, aux3_h, *rest):
        if mode == 1:
            h_out, sd_out, ss_out = rest[:3]
            rest = rest[3:]
        else:
            hd_out, a_out = rest[:2]
            rest = rest[2:]
        (T32, ssrc_v, sdst_v, res_v, resi_v, sidx_v, didx_v, ef_v,
         rows0, rows1, stage0, stage1, tbuf, zbuf, obuf, sa_v, sb_v,
         consts_v, aux1_v, aux2_v, aux3_v,
         sem, sem_g0, sem_g1, sem_sc0, sem_sc1) = rest
        rows_b = (rows0, rows1)
        stage_b = (stage0, stage1)
        sem_g = (sem_g0, sem_g1)
        sem_sc = (sem_sc0, sem_sc1)

        c = lax.axis_index("c")
        s = lax.axis_index("s")

        pltpu.sync_copy(ssrcT.at[pl.ds(c * n_src, n_src)], ssrc_v)
        pltpu.sync_copy(sdstT.at[pl.ds(c * n_src, n_src)], sdst_v)
        pltpu.sync_copy(resf_h, res_v)
        pltpu.sync_copy(resi_h, resi_v)
        pltpu.sync_copy(consts_h, consts_v)
        pltpu.sync_copy(aux1_h, aux1_v)
        pltpu.sync_copy(aux2_h, aux2_v)
        pltpu.sync_copy(aux3_h, aux3_v)

        zero16 = jnp.zeros((16,), F32)

        @plsc.parallel_loop(0, NPT)
        def zrow(i):
            tbuf[i, pl.ds(0, 16)] = zero16
            tbuf[i, pl.ds(16, 16)] = zero16

        pltpu.sync_copy(tbuf, T32.at[pl.ds(s * NPT, NPT)])
        plsc.subcore_barrier()

        cv = consts_v[...]
        cvec = jnp.full((16,), cv[0])
        gbvec = jnp.full((16,), cv[1])
        lane = lax.iota(I32, 16)
        e1 = jnp.where(lane == 0, 1.0, 0.0).astype(F32)

        @plsc.parallel_loop(0, K)
        def initstage(e):
            stage0[e, pl.ds(16, 16)] = e1
            stage1[e, pl.ds(16, 16)] = e1

        el = [pltpu.async_copy(src1_h.at[pl.ds(s * EPT, EPT)], sidx_v, sem),
              pltpu.async_copy(dst1_h.at[pl.ds(s * EPT, EPT)], didx_v, sem),
              pltpu.async_copy(ef_h.at[pl.ds(s * EPT, EPT)], ef_v, sem)]
        for d in el:
            d.wait()

        def g_pairs(g, p):
            return [(V_h.at[c].at[sidx_v.at[pl.ds(g * K + j * SUB, SUB)]],
                     rows_b[p].at[pl.ds(j * SUB, SUB)])
                    for j in range(NSUB)]

        def sc_pairs(g, p):
            return [(stage_b[p].at[pl.ds(j * SUB, SUB)],
                     T32.at[didx_v.at[pl.ds(g * K + j * SUB, SUB)]])
                    for j in range(NSUB)]

        def fire_g(g, p):
            for src, dst in g_pairs(g, p):
                pltpu.async_copy(src, dst, sem_g[p])

        def wait_g(g, p):
            for src, dst in g_pairs(g, p):
                pltpu.make_async_copy(src, dst, sem_g[p]).wait()

        def fire_sc(g, p):
            for src, dst in sc_pairs(g, p):
                pltpu.async_copy(src, dst, sem_sc[p], add=True)

        def wait_sc(g, p):
            for src, dst in sc_pairs(g, p):
                pltpu.make_async_copy(src, dst, sem_sc[p]).wait()

        def compute(g, p):
            rows_p = rows_b[p]
            stage_p = stage_b[p]

            @plsc.parallel_loop(0, NGATE, unroll=4)
            def gate(v):
                off = g * K + v * 16
                si = sidx_v[pl.ds(off, 16)]
                di = didx_v[pl.ds(off, 16)]
                efv = ef_v[pl.ds(off, 16)]
                nid = plsc.bitcast(plsc.load_gather(res_v, [di]), I32)
                a = (plsc.load_gather(ssrc_v, [si])
                     + plsc.load_gather(sdst_v, [nid])
                     + efv * cvec + gbvec)
                lam = 1.0 / (1.0 + jnp.exp(-a))
                g16 = efv * lam
                for j in range(16):
                    e = v * 16 + j
                    gv = jnp.full((16,), g16[j])
                    stage_p[e, pl.ds(0, 16)] = rows_p[e, :] * gv

        def one(g, p, first):
            if not first:
                wait_sc(g - 2, p)
            wait_g(g, p)
            cnext = jnp.where(g + 1 < NCHUNK, g + 1, 0)
            fire_g(cnext, 1 - p)
            compute(g, p)
            fire_sc(g, p)

        fire_g(0, 0)
        one(0, 0, True)
        one(1, 1, True)

        def pair(m, carry):
            g = 2 + 2 * m
            one(g, 0, False)
            one(g + 1, 1, False)
            return carry

        lax.fori_loop(0, (NCHUNK - 2) // 2, pair, 0)
        wait_sc(NCHUNK - 2, 0)
        wait_sc(NCHUNK - 1, 1)
        wait_g(0, 0)
        plsc.subcore_barrier()

        nb = s * NPT
        pltpu.sync_copy(T32.at[pl.ds(nb, NPT)], tbuf)
        zs = [pltpu.async_copy(
                  Z_h.at[c].at[resi_v.at[pl.ds(nb + j * SUB, SUB)]],
                  zbuf.at[pl.ds(j * SUB, SUB)], sem)
              for j in range(NPT // SUB)]
        for z in zs:
            z.wait()

        if mode == 1:
            bvec = aux1_v[...]
            gwd = aux2_v[...]
            gws = aux3_v[...]

            @plsc.parallel_loop(0, NPT // 16, unroll=2)
            def node(grp):
                base = grp * 16
                acc_a = jnp.zeros((16,), F32)
                acc_b = jnp.zeros((16,), F32)
                for j in range(16):
                    i = base + j
                    rcv = 1.0 / jnp.maximum(tbuf[i, pl.ds(16, 16)], 1.0)
                    rc = jnp.full((16,), rcv[0])
                    hrow = zbuf[i, :] + tbuf[i, pl.ds(0, 16)] * rc + bvec
                    hrow = jnp.maximum(hrow, hrow * 0.01)
                    obuf[i, :] = hrow
                    acc_a = jnp.where(lane == j, jnp.sum(hrow * gwd), acc_a)
                    acc_b = jnp.where(lane == j, jnp.sum(hrow * gws), acc_b)
                sa_v[pl.ds(base, 16)] = acc_a
                sb_v[pl.ds(base, 16)] = acc_b
            pltpu.sync_copy(obuf, h_out.at[c].at[pl.ds(nb, NPT)])
            pltpu.sync_copy(sa_v, sd_out.at[pl.ds(c * n_dst_pad + nb, NPT)])
            pltpu.sync_copy(sb_v, ss_out.at[pl.ds(c * n_dst_pad + nb, NPT)])
        else:
            ampv = aux1_v[...]

            @plsc.parallel_loop(0, NPT // 16, unroll=2)
            def node(grp):
                base = grp * 16
                for j in range(16):
                    i = base + j
                    rcv = 1.0 / jnp.maximum(tbuf[i, pl.ds(16, 16)], 1.0)
                    rc = jnp.full((16,), rcv[0])
                    obuf[i, :] = ampv * tbuf[i, pl.ds(0, 16)] * rc
            pltpu.sync_copy(zbuf, hd_out.at[c].at[pl.ds(nb, NPT)])
            pltpu.sync_copy(obuf, a_out.at[c].at[pl.ds(nb, NPT)])

    return pl.kernel(
        body, out_type=out_type, mesh=mesh, scratch_types=scratch,
        compiler_params=pltpu.CompilerParams(
            needs_layout_passes=False, use_tc_tiling_on_sc=False))


def _fin_body(hd_ref, a_ref, w2_ref, b2_ref, o_ref):
    hd = hd_ref[0]
    a = a_ref[0]
    w2 = w2_ref[...]
    o_ref[0] = (jnp.dot(hd, w2[:CH], preferred_element_type=F32)
                + jnp.dot(a, w2[CH:], preferred_element_type=F32)
                + b2_ref[...])


@jax.jit
def _fin(hd, A2, W2, b2r):
    return pl.pallas_call(
        _fin_body,
        grid=(B,),
        in_specs=[
            pl.BlockSpec((1, N2P, CH), lambda b: (b, 0, 0)),
            pl.BlockSpec((1, N2P, CH), lambda b: (b, 0, 0)),
            pl.BlockSpec((2 * CH, C_IN), lambda b: (0, 0)),
            pl.BlockSpec((1, C_IN), lambda b: (0, 0)),
        ],
        out_specs=pl.BlockSpec((1, N2P, C_IN), lambda b: (b, 0, 0)),
        out_shape=jax.ShapeDtypeStruct((B, N2P, C_IN), F32),
    )(hd, A2, W2, b2r)


_sc_layer1 = _make_sc_layer(1, N0P, N1P, E0P)
_sc_layer2 = _make_sc_layer(2, B * N1P // 2, N2P, E1P)


def _vec16(*vals):
    v = jnp.zeros((16,), F32)
    for i, x in enumerate(vals):
        v = v.at[i].set(x)
    return v


@jax.jit
def kernel(X, edge_weight0, edge_weight1, W1, b1, amp1, gW1, gb1,
           W2, b2, amp2, gW2, gb2, edge_index0, edge_index1,
           res_n_id0, res_n_id1):
    src0, dst0 = edge_index0[0], edge_index0[1]
    src1, dst1 = edge_index1[0], edge_index1[1]

    Z1, U1, ssrc1, sdstf1 = _pre(X, W1, amp1, gW1[:, 0][None, :])
    ssrc1 = jnp.pad(ssrc1.reshape(B, N0), ((0, 0), (0, N0P - N0))).reshape(-1)
    sdstf1 = jnp.pad(sdstf1.reshape(B, N0), ((0, 0), (0, N0P - N0))).reshape(-1)

    res0p = jnp.pad(res_n_id0, (0, N1P - N1))
    res1p = jnp.pad(res_n_id1, (0, N2P - N2))
    src0p = jnp.pad(src0, (0, E0P - E0))
    dst0p = jnp.pad(dst0, (0, E0P - E0), constant_values=N1)
    ew0p = jnp.pad(edge_weight0, (0, E0P - E0))
    src1p = jnp.pad(src1, (0, E1P - E1))
    dst1p = jnp.pad(dst1, (0, E1P - E1), constant_values=N2)
    ew1p = jnp.pad(edge_weight1, (0, E1P - E1))

    consts1 = _vec16(gW1[2 * C_IN, 0], gb1[0])
    consts2 = _vec16(gW2[2 * CH, 0], gb2[0])

    res0f = lax.bitcast_convert_type(res0p, F32)
    res1f = lax.bitcast_convert_type(res1p, F32)

    h, sdst2, ssrc2 = _sc_layer1(
        U1, Z1, ssrc1, sdstf1, res0f, res0p, src0p, dst0p, ew0p,
        consts1, b1, gW2[:CH, 0], gW2[CH:2 * CH, 0])

    hd, A2 = _sc_layer2(
        h, h, ssrc2, sdst2, res1f, res1p, src1p, dst1p, ew1p,
        consts2, amp2[0], consts2, consts2)

    out = _fin(hd, A2, W2, b2[None, :])
    return out[:, :N2]


# speedup vs baseline: 1.0057x; 1.0057x over previous
import jax
import jax.numpy as jnp
from jax import lax
from jax.experimental import pallas as pl
from jax.experimental.pallas import tpu as pltpu
from jax.experimental.pallas import tpu_sc as plsc

B, N0, C_IN = 2, 10000, 128
N1, N2 = 5000, 2500
E0, E1 = 160000, 80000
CH = 16
SUB = 128
K = 256
N0P = 10240
N1P, N2P = 6144, 4096
E0P, E1P = 163840, 81920

F32 = jnp.float32
I32 = jnp.int32


def _pre_body(x_ref, w1_ref, amp_ref, gw_ref, z_ref, u_ref, ssrc_ref, sdst_ref):
    x = x_ref[0]
    w1 = w1_ref[...]
    amp = amp_ref[...]
    gw = gw_ref[...]
    z_ref[0] = jnp.dot(x, w1[:C_IN], preferred_element_type=F32)
    u_ref[0] = jnp.dot(x * amp, w1[C_IN:], preferred_element_type=F32)
    ssrc_ref[0, 0] = jnp.sum(x * gw[0, C_IN:2 * C_IN][None, :], axis=1)
    sdst_ref[0, 0] = jnp.sum(x * gw[0, 0:C_IN][None, :], axis=1)


@jax.jit
def _pre(X, W1, amp1, gw1r):
    return pl.pallas_call(
        _pre_body,
        grid=(B,),
        in_specs=[
            pl.BlockSpec((1, N0, C_IN), lambda b: (b, 0, 0)),
            pl.BlockSpec((2 * C_IN, CH), lambda b: (0, 0)),
            pl.BlockSpec((1, C_IN), lambda b: (0, 0)),
            pl.BlockSpec((1, 2 * C_IN + 1), lambda b: (0, 0)),
        ],
        out_specs=[
            pl.BlockSpec((1, N0, CH), lambda b: (b, 0, 0)),
            pl.BlockSpec((1, N0, CH), lambda b: (b, 0, 0)),
            pl.BlockSpec((1, 1, N0), lambda b: (b, 0, 0)),
            pl.BlockSpec((1, 1, N0), lambda b: (b, 0, 0)),
        ],
        out_shape=[
            jax.ShapeDtypeStruct((B, N0, CH), F32),
            jax.ShapeDtypeStruct((B, N0, CH), F32),
            jax.ShapeDtypeStruct((B, 1, N0), F32),
            jax.ShapeDtypeStruct((B, 1, N0), F32),
        ],
    )(X, W1, amp1, gw1r)


def _make_sc_layer(mode, n_src, n_dst_pad, e_total):
    NPT = n_dst_pad // 16
    EPT = e_total // 16
    NCHUNK = EPT // K
    NSUB = K // SUB
    NGATE = K // 16
    mesh = plsc.VectorSubcoreMesh(core_axis_name="c", subcore_axis_name="s")

    if mode == 1:
        out_type = [
            jax.ShapeDtypeStruct((B, n_dst_pad, CH), F32),
            jax.ShapeDtypeStruct((B * n_dst_pad,), F32),
            jax.ShapeDtypeStruct((B * n_dst_pad,), F32),
        ]
    else:
        out_type = [
            jax.ShapeDtypeStruct((B, n_dst_pad, CH), F32),
            jax.ShapeDtypeStruct((B, n_dst_pad, CH), F32),
        ]

    scratch = [
        pltpu.VMEM_SHARED((n_dst_pad, CH), F32),
        pltpu.VMEM_SHARED((n_dst_pad, CH), F32),
        pltpu.VMEM((n_src,), F32),
        pltpu.VMEM((n_src,), F32),
        pltpu.VMEM((n_dst_pad,), F32),
        pltpu.VMEM((n_dst_pad,), I32),
        pltpu.VMEM((EPT,), I32),
        pltpu.VMEM((EPT,), I32),
        pltpu.VMEM((EPT,), F32),
        pltpu.VMEM((K, CH), F32),
        pltpu.VMEM((K, CH), F32),
        pltpu.VMEM((K, CH), F32),
        pltpu.VMEM((K, CH), F32),
        pltpu.VMEM((K, CH), F32),
        pltpu.VMEM((NPT, CH), F32),
        pltpu.VMEM((NPT, CH), F32),
        pltpu.VMEM((NPT, CH), F32),
        pltpu.VMEM((NPT, CH), F32),
        pltpu.VMEM((NPT,), F32),
        pltpu.VMEM((NPT,), F32),
        pltpu.VMEM((16,), F32),
        pltpu.VMEM((16,), F32),
        pltpu.VMEM((16,), F32),
        pltpu.VMEM((16,), F32),
        pltpu.SemaphoreType.DMA,
        pltpu.SemaphoreType.DMA,
        pltpu.SemaphoreType.DMA,
        pltpu.SemaphoreType.DMA,
        pltpu.SemaphoreType.DMA,
    ]

    def body(V_h, Z_h, ssrcT, sdstT, resf_h, resi_h, src1_h, dst1_h, ef_h,
             consts_h, aux1_h, aux2_h, aux3_h, *rest):
        if mode == 1:
            h_out, sd_out, ss_out = rest[:3]
            rest = rest[3:]
        else:
            hd_out, a_out = rest[:2]
            rest = rest[2:]
        (T16, C16, ssrc_v, sdst_v, res_v, resi_v, sidx_v, didx_v, ef_v,
         rows0, rows1, stage0, stage1, ones_v, tbuf, cbuf, zbuf, obuf,
         sa_v, sb_v, consts_v, aux1_v, aux2_v, aux3_v,
         sem, sem_g0, sem_g1, sem_sc0, sem_sc1) = rest
        rows_b = (rows0, rows1)
        stage_b = (stage0, stage1)
        sem_g = (sem_g0, sem_g1)
        sem_sc = (sem_sc0, sem_sc1)

        c = lax.axis_index("c")
        s = lax.axis_index("s")

        pltpu.sync_copy(ssrcT.at[pl.ds(c * n_src, n_src)], ssrc_v)
        pltpu.sync_copy(sdstT.at[pl.ds(c * n_src, n_src)], sdst_v)
        pltpu.sync_copy(resf_h, res_v)
        pltpu.sync_copy(resi_h, resi_v)
        pltpu.sync_copy(consts_h, consts_v)
        pltpu.sync_copy(aux1_h, aux1_v)
        pltpu.sync_copy(aux2_h, aux2_v)
        pltpu.sync_copy(aux3_h, aux3_v)

        zero16 = jnp.zeros((16,), F32)

        @plsc.parallel_loop(0, NPT)
        def zrow(i):
            tbuf[i, :] = zero16

        pltpu.sync_copy(tbuf, T16.at[pl.ds(s * NPT, NPT)])
        pltpu.sync_copy(tbuf, C16.at[pl.ds(s * NPT, NPT)])
        plsc.subcore_barrier()

        cv = consts_v[...]
        cvec = jnp.full((16,), cv[0])
        gbvec = jnp.full((16,), cv[1])
        lane = lax.iota(I32, 16)
        e1 = jnp.where(lane == 0, 1.0, 0.0).astype(F32)

        @plsc.parallel_loop(0, K)
        def initones(e):
            ones_v[e, :] = e1

        el = [pltpu.async_copy(src1_h.at[pl.ds(s * EPT, EPT)], sidx_v, sem),
              pltpu.async_copy(dst1_h.at[pl.ds(s * EPT, EPT)], didx_v, sem),
              pltpu.async_copy(ef_h.at[pl.ds(s * EPT, EPT)], ef_v, sem)]
        for d in el:
            d.wait()

        def g_pairs(g, p):
            return [(V_h.at[c].at[sidx_v.at[pl.ds(g * K + j * SUB, SUB)]],
                     rows_b[p].at[pl.ds(j * SUB, SUB)])
                    for j in range(NSUB)]

        def sc_pairs(g, p):
            prs = []
            for j in range(NSUB):
                idx = didx_v.at[pl.ds(g * K + j * SUB, SUB)]
                prs.append((stage_b[p].at[pl.ds(j * SUB, SUB)], T16.at[idx]))
                prs.append((ones_v.at[pl.ds(j * SUB, SUB)], C16.at[idx]))
            return prs

        def fire_g(g, p):
            for src, dst in g_pairs(g, p):
                pltpu.async_copy(src, dst, sem_g[p])

        def wait_g(g, p):
            for src, dst in g_pairs(g, p):
                pltpu.make_async_copy(src, dst, sem_g[p]).wait()

        def fire_sc(g, p):
            for src, dst in sc_pairs(g, p):
                pltpu.async_copy(src, dst, sem_sc[p], add=True)

        def wait_sc(g, p):
            for src, dst in sc_pairs(g, p):
                pltpu.make_async_copy(src, dst, sem_sc[p]).wait()

        def compute(g, p):
            rows_p = rows_b[p]
            stage_p = stage_b[p]

            @plsc.parallel_loop(0, NGATE, unroll=4)
            def gate(v):
                off = g * K + v * 16
                si = sidx_v[pl.ds(off, 16)]
                di = didx_v[pl.ds(off, 16)]
                efv = ef_v[pl.ds(off, 16)]
                nid = plsc.bitcast(plsc.load_gather(res_v, [di]), I32)
                a = (plsc.load_gather(ssrc_v, [si])
                     + plsc.load_gather(sdst_v, [nid])
                     + efv * cvec + gbvec)
                lam = 1.0 / (1.0 + jnp.exp(-a))
                g16 = efv * lam
                for j in range(16):
                    e = v * 16 + j
                    gv = jnp.full((16,), g16[j])
                    stage_p[e, :] = rows_p[e, :] * gv

        def one(g, p, first):
            if not first:
                wait_sc(g - 2, p)
            wait_g(g, p)
            cnext = jnp.where(g + 1 < NCHUNK, g + 1, 0)
            fire_g(cnext, 1 - p)
            compute(g, p)
            fire_sc(g, p)

        fire_g(0, 0)
        one(0, 0, True)
        one(1, 1, True)

        def pair(m, carry):
            g = 2 + 2 * m
            one(g, 0, False)
            one(g + 1, 1, False)
            return carry

        lax.fori_loop(0, (NCHUNK - 2) // 2, pair, 0)
        wait_sc(NCHUNK - 2, 0)
        wait_sc(NCHUNK - 1, 1)
        wait_g(0, 0)
        plsc.subcore_barrier()

        nb = s * NPT
        pltpu.sync_copy(T16.at[pl.ds(nb, NPT)], tbuf)
        pltpu.sync_copy(C16.at[pl.ds(nb, NPT)], cbuf)
        zs = [pltpu.async_copy(
                  Z_h.at[c].at[resi_v.at[pl.ds(nb + j * SUB, SUB)]],
                  zbuf.at[pl.ds(j * SUB, SUB)], sem)
              for j in range(NPT // SUB)]
        for z in zs:
            z.wait()

        if mode == 1:
            bvec = aux1_v[...]
            gwd = aux2_v[...]
            gws = aux3_v[...]

            @plsc.parallel_loop(0, NPT // 16, unroll=2)
            def node(grp):
                base = grp * 16
                acc_a = jnp.zeros((16,), F32)
                acc_b = jnp.zeros((16,), F32)
                for j in range(16):
                    i = base + j
                    rcv = 1.0 / jnp.maximum(cbuf[i, :], 1.0)
                    rc = jnp.full((16,), rcv[0])
                    hrow = zbuf[i, :] + tbuf[i, :] * rc + bvec
                    hrow = jnp.maximum(hrow, hrow * 0.01)
                    obuf[i, :] = hrow
                    acc_a = jnp.where(lane == j, jnp.sum(hrow * gwd), acc_a)
                    acc_b = jnp.where(lane == j, jnp.sum(hrow * gws), acc_b)
                sa_v[pl.ds(base, 16)] = acc_a
                sb_v[pl.ds(base, 16)] = acc_b
            pltpu.sync_copy(obuf, h_out.at[c].at[pl.ds(nb, NPT)])
            pltpu.sync_copy(sa_v, sd_out.at[pl.ds(c * n_dst_pad + nb, NPT)])
            pltpu.sync_copy(sb_v, ss_out.at[pl.ds(c * n_dst_pad + nb, NPT)])
        else:
            ampv = aux1_v[...]

            @plsc.parallel_loop(0, NPT // 16, unroll=2)
            def node(grp):
                base = grp * 16
                for j in range(16):
                    i = base + j
                    rcv = 1.0 / jnp.maximum(cbuf[i, :], 1.0)
                    rc = jnp.full((16,), rcv[0])
                    obuf[i, :] = ampv * tbuf[i, :] * rc
            pltpu.sync_copy(zbuf, hd_out.at[c].at[pl.ds(nb, NPT)])
            pltpu.sync_copy(obuf, a_out.at[c].at[pl.ds(nb, NPT)])

    return pl.kernel(
        body, out_type=out_type, mesh=mesh, scratch_types=scratch,
        compiler_params=pltpu.CompilerParams(
            needs_layout_passes=False, use_tc_tiling_on_sc=False))


def _fin_body(hd_ref, a_ref, w2_ref, b2_ref, o_ref):
    hd = hd_ref[0]
    a = a_ref[0]
    w2 = w2_ref[...]
    o_ref[0] = (jnp.dot(hd, w2[:CH], preferred_element_type=F32)
                + jnp.dot(a, w2[CH:], preferred_element_type=F32)
                + b2_ref[...])


@jax.jit
def _fin(hd, A2, W2, b2r):
    return pl.pallas_call(
        _fin_body,
        grid=(B,),
        in_specs=[
            pl.BlockSpec((1, N2P, CH), lambda b: (b, 0, 0)),
            pl.BlockSpec((1, N2P, CH), lambda b: (b, 0, 0)),
            pl.BlockSpec((2 * CH, C_IN), lambda b: (0, 0)),
            pl.BlockSpec((1, C_IN), lambda b: (0, 0)),
        ],
        out_specs=pl.BlockSpec((1, N2P, C_IN), lambda b: (b, 0, 0)),
        out_shape=jax.ShapeDtypeStruct((B, N2P, C_IN), F32),
    )(hd, A2, W2, b2r)


_sc_layer1 = _make_sc_layer(1, N0P, N1P, E0P)
_sc_layer2 = _make_sc_layer(2, B * N1P // 2, N2P, E1P)


def _vec16(*vals):
    v = jnp.zeros((16,), F32)
    for i, x in enumerate(vals):
        v = v.at[i].set(x)
    return v


@jax.jit
def kernel(X, edge_weight0, edge_weight1, W1, b1, amp1, gW1, gb1,
           W2, b2, amp2, gW2, gb2, edge_index0, edge_index1,
           res_n_id0, res_n_id1):
    src0, dst0 = edge_index0[0], edge_index0[1]
    src1, dst1 = edge_index1[0], edge_index1[1]

    Z1, U1, ssrc1, sdstf1 = _pre(X, W1, amp1, gW1[:, 0][None, :])
    ssrc1 = jnp.pad(ssrc1.reshape(B, N0), ((0, 0), (0, N0P - N0))).reshape(-1)
    sdstf1 = jnp.pad(sdstf1.reshape(B, N0), ((0, 0), (0, N0P - N0))).reshape(-1)

    res0p = jnp.pad(res_n_id0, (0, N1P - N1))
    res1p = jnp.pad(res_n_id1, (0, N2P - N2))
    src0p = jnp.pad(src0, (0, E0P - E0))
    dst0p = jnp.pad(dst0, (0, E0P - E0), constant_values=N1)
    ew0p = jnp.pad(edge_weight0, (0, E0P - E0))
    src1p = jnp.pad(src1, (0, E1P - E1))
    dst1p = jnp.pad(dst1, (0, E1P - E1), constant_values=N2)
    ew1p = jnp.pad(edge_weight1, (0, E1P - E1))

    consts1 = _vec16(gW1[2 * C_IN, 0], gb1[0])
    consts2 = _vec16(gW2[2 * CH, 0], gb2[0])

    res0f = lax.bitcast_convert_type(res0p, F32)
    res1f = lax.bitcast_convert_type(res1p, F32)

    h, sdst2, ssrc2 = _sc_layer1(
        U1, Z1, ssrc1, sdstf1, res0f, res0p, src0p, dst0p, ew0p,
        consts1, b1, gW2[:CH, 0], gW2[CH:2 * CH, 0])

    hd, A2 = _sc_layer2(
        h, h, ssrc2, sdst2, res1f, res1p, src1p, dst1p, ew1p,
        consts2, amp2[0], consts2, consts2)

    out = _fin(hd, A2, W2, b2[None, :])
    return out[:, :N2]
